# baseline (speedup 1.0000x reference)
import functools

import jax
import jax.numpy as jnp
from jax import lax
from jax.experimental import pallas as pl
from jax.experimental.pallas import tpu as pltpu
from jax.experimental.pallas import tpu_sc as plsc

NUM_RADIAL = 128
NUM_SPH = 8
EMB = 32
OUT_COLS = NUM_SPH * EMB

_NC = 2
_NS = 16
_NW = _NC * _NS
_CHUNK = 128
_NBUF = 4


def _sc_gather(table, idx, t_pad, nchunks):
    mesh = plsc.VectorSubcoreMesh(core_axis_name="c", subcore_axis_name="s")

    @functools.partial(
        pl.kernel,
        mesh=mesh,
        out_type=jax.ShapeDtypeStruct((t_pad, NUM_RADIAL), jnp.float32),
        scratch_types=[
            pltpu.VMEM((nchunks, _CHUNK), jnp.int32),
            pltpu.VMEM((_NBUF, _CHUNK, NUM_RADIAL), jnp.float32),
            pltpu.SemaphoreType.DMA,
            pltpu.SemaphoreType.DMA,
            pltpu.SemaphoreType.DMA,
            pltpu.SemaphoreType.DMA,
            pltpu.SemaphoreType.DMA,
            pltpu.SemaphoreType.DMA,
            pltpu.SemaphoreType.DMA,
            pltpu.SemaphoreType.DMA,
        ],
    )
    def k(table_hbm, idx_hbm, out_hbm, idx_v, rows_v, *sems):
        gsem, wsem = sems[:_NBUF], sems[_NBUF:]
        wid = lax.axis_index("s") * _NC + lax.axis_index("c")
        base = wid * nchunks
        pltpu.sync_copy(idx_hbm.at[wid], idx_v)

        def gather(c, b):
            return pltpu.make_async_copy(
                table_hbm.at[idx_v.at[c]], rows_v.at[b], gsem[b])

        def wback(c, b):
            return pltpu.make_async_copy(
                rows_v.at[b],
                out_hbm.at[pl.ds((base + c) * _CHUNK, _CHUNK)], wsem[b])

        for b in range(_NBUF):
            gather(b, b).start()

        def body(j, carry):
            c0 = _NBUF * j
            for b in range(_NBUF):
                gather(c0 + b, b).wait()
                wback(c0 + b, b).start()
            for b in range(_NBUF):
                wback(c0 + b, b).wait()
                gather(c0 + _NBUF + b, b).start()
            return carry

        lax.fori_loop(0, nchunks // _NBUF - 1, body, 0, unroll=False)
        c0 = nchunks - _NBUF
        for b in range(_NBUF):
            gather(c0 + b, b).wait()
            wback(c0 + b, b).start()
        for b in range(_NBUF):
            wback(c0 + b, b).wait()

    return k(table, idx)


def _tc_contract(g, sph, w, b_mat, p_mat, t_pad, tile):
    def body(g_ref, s_ref, w_ref, b_ref, p_ref, o_ref):
        h = jnp.dot(g_ref[...], w_ref[...], preferred_element_type=jnp.float32)
        srep = jnp.dot(s_ref[...], b_ref[...], preferred_element_type=jnp.float32)
        o_ref[...] = jnp.dot(h * srep, p_ref[...],
                             preferred_element_type=jnp.float32)

    return pl.pallas_call(
        body,
        grid=(t_pad // tile,),
        in_specs=[
            pl.BlockSpec((tile, NUM_RADIAL), lambda i: (i, 0)),
            pl.BlockSpec((tile, NUM_SPH), lambda i: (i, 0)),
            pl.BlockSpec((NUM_RADIAL, OUT_COLS), lambda i: (0, 0)),
            pl.BlockSpec((NUM_SPH, OUT_COLS), lambda i: (0, 0)),
            pl.BlockSpec((OUT_COLS, EMB), lambda i: (0, 0)),
        ],
        out_specs=pl.BlockSpec((tile, EMB), lambda i: (i, 0)),
        out_shape=jax.ShapeDtypeStruct((t_pad, EMB), jnp.float32),
    )(g, sph, w, b_mat, p_mat)


def kernel(rbf, sph, idx_sph, weight):
    t = idx_sph.shape[0]
    tile = 1024
    per_w = -(-t // (_NW * _CHUNK)) * _CHUNK
    nchunks = per_w // _CHUNK
    t_pad = _NW * per_w
    while nchunks % _NBUF or (_NW * nchunks * _CHUNK) % tile:
        nchunks += 1
    t_pad = _NW * nchunks * _CHUNK

    idx_pad = jnp.zeros((t_pad,), jnp.int32).at[:t].set(idx_sph)
    sph_pad = jnp.zeros((t_pad, NUM_SPH), sph.dtype).at[:t].set(sph)

    g = _sc_gather(rbf, idx_pad.reshape(_NW, nchunks, _CHUNK), t_pad, nchunks)

    w = weight.reshape(NUM_RADIAL, OUT_COLS)
    b_mat = jnp.tile(jnp.eye(NUM_SPH, dtype=jnp.float32), (1, EMB))
    p_mat = jnp.repeat(jnp.eye(EMB, dtype=jnp.float32), NUM_SPH, axis=0)

    out = _tc_contract(g, sph_pad, w, b_mat, p_mat, t_pad, tile)
    return out[:t]
